# Initial kernel scaffold; baseline (speedup 1.0000x reference)
import jax
import jax.numpy as jnp
from jax.experimental import pallas as pl


def kernel(input, W, P, gamma, beta):
    raise NotImplementedError("write your pallas kernel here")


# speedup vs baseline: 1.9087x; 1.9087x over previous
import functools

import jax
import jax.numpy as jnp
from jax import lax
from jax.experimental import pallas as pl
from jax.experimental.pallas import tpu as pltpu
from jax.experimental.pallas import tpu_sc as plsc

VOC = 100000
DIM = 64
MAXLEN = 200
B = 4096
L = 200

NC = 2
NS = 16
LANES = 16
NW = NC * NS
ROWS = B * L
ROWS_PER_W = ROWS // NW
CHUNK = 100
NCHUNK = ROWS_PER_W // CHUNK
NVEC = DIM // LANES


def _rsqrt(x):
    bits = lax.bitcast_convert_type(x, jnp.int32)
    y = lax.bitcast_convert_type(jnp.int32(0x5F3759DF) - (bits >> 1), jnp.float32)
    for _ in range(3):
        y = y * (1.5 - 0.5 * x * y * y)
    return y


def _emb_body(idx_hbm, w_hbm, p_hbm, gamma_hbm, beta_hbm, out_hbm,
              idx_v, rows_v, p_v, g_v, b_v, sem0, sem1):
    wid = lax.axis_index("s") * NC + lax.axis_index("c")

    pltpu.sync_copy(idx_hbm.at[wid], idx_v)
    pltpu.sync_copy(p_hbm, p_v)
    pltpu.sync_copy(gamma_hbm, g_v)
    pltpu.sync_copy(beta_hbm, b_v)

    g = [g_v[pl.ds(16 * j, 16)] for j in range(NVEC)]
    bta = [b_v[pl.ds(16 * j, 16)] for j in range(NVEC)]
    sems = (sem0, sem1)

    def gather(c, buf):
        return pltpu.make_async_copy(
            w_hbm.at[idx_v.at[c]], rows_v.at[buf], sems[buf])

    def compute(buf, p_off):
        def row_body(i, _):
            w = [rows_v[buf, i, pl.ds(16 * j, 16)] for j in range(NVEC)]
            p = [p_v[p_off + i, pl.ds(16 * j, 16)] for j in range(NVEC)]
            v = [w[j] + p[j] for j in range(NVEC)]
            acc = (v[0] + v[1]) + (v[2] + v[3])
            acc2 = ((v[0] * v[0] + v[1] * v[1])
                    + (v[2] * v[2] + v[3] * v[3]))
            s1 = jnp.broadcast_to(jnp.sum(acc), (LANES,))
            s2 = jnp.broadcast_to(jnp.sum(acc2), (LANES,))
            mean = s1 * (1.0 / DIM)
            var = s2 * (1.0 / DIM) - mean * mean
            r = _rsqrt(var + 1e-5)
            for j in range(NVEC):
                rows_v[buf, i, pl.ds(16 * j, 16)] = (
                    (v[j] - mean) * r * g[j] + bta[j])
            return _
        lax.fori_loop(0, CHUNK, row_body, None)

    gather(0, 0).start()

    def pair_body(t, _):
        c0 = 2 * t
        gather(c0 + 1, 1).start()
        gather(c0, 0).wait()
        compute(0, 0)
        pltpu.sync_copy(rows_v.at[0], out_hbm.at[wid, c0])

        @pl.when(c0 + 2 < NCHUNK)
        def _fire():
            gather(c0 + 2, 0).start()

        gather(c0 + 1, 1).wait()
        compute(1, CHUNK)
        pltpu.sync_copy(rows_v.at[1], out_hbm.at[wid, c0 + 1])
        return _

    lax.fori_loop(0, NCHUNK // 2, pair_body, None)


@jax.jit
def _emb_kernel(idx, w, p, gamma, beta):
    mesh = plsc.VectorSubcoreMesh(core_axis_name="c", subcore_axis_name="s")
    f = functools.partial(
        pl.kernel,
        mesh=mesh,
        compiler_params=pltpu.CompilerParams(
            needs_layout_passes=False, use_tc_tiling_on_sc=False),
        out_type=jax.ShapeDtypeStruct((NW, NCHUNK, CHUNK, DIM), jnp.float32),
        scratch_types=[
            pltpu.VMEM((NCHUNK, CHUNK), jnp.int32),
            pltpu.VMEM((2, CHUNK, DIM), jnp.float32),
            pltpu.VMEM((MAXLEN, DIM), jnp.float32),
            pltpu.VMEM((DIM,), jnp.float32),
            pltpu.VMEM((DIM,), jnp.float32),
            pltpu.SemaphoreType.DMA,
            pltpu.SemaphoreType.DMA,
        ],
    )(_emb_body)
    return f(idx, w, p, gamma, beta)


def kernel(input, W, P, gamma, beta):
    idx = input.reshape(NW, NCHUNK, CHUNK).astype(jnp.int32)
    out = _emb_kernel(idx, W, P, gamma, beta)
    return out.reshape(B, L, DIM)
